# baseline (speedup 1.0000x reference)
import functools

import jax
import jax.numpy as jnp
from jax.experimental import pallas as pl
from jax.experimental.pallas import tpu as pltpu


def _round_up(x, m):
    return ((x + m - 1) // m) * m


def _head_kernel(x_ref, w1_ref, b1_ref, w2_ref, b2_ref, out_ref, *, inv_hw):
    x = x_ref[...]
    pooled = jnp.sum(x, axis=1) * inv_hw
    h = jax.lax.dot_general(
        pooled, w1_ref[...], (((1,), (1,)), ((), ())),
        preferred_element_type=jnp.float32) + b1_ref[...]
    out = jax.lax.dot_general(
        h, w2_ref[...], (((1,), (1,)), ((), ())),
        preferred_element_type=jnp.float32) + b2_ref[...]
    out_ref[...] = out.astype(out_ref.dtype)


def kernel(x, w1, b1, w2, b2):
    B, C, H, W = x.shape
    hidden = w1.shape[0]
    NC = w2.shape[0]
    HW = H * W

    TB = 8
    B_pad = _round_up(max(B, TB), TB)

    xr = jnp.transpose(x, (0, 2, 3, 1)).reshape(B, HW, C)
    if B_pad != B:
        xr = jnp.pad(xr, ((0, B_pad - B), (0, 0), (0, 0)))

    b1_row = b1.reshape(1, hidden)
    b2_row = b2.reshape(1, NC)

    n_b = B_pad // TB
    x_tile_bytes = TB * _round_up(HW, 8) * _round_up(C, 128) * 4
    weight_bytes = (hidden * _round_up(C, 128) + _round_up(hidden, 128)
                    + NC * _round_up(hidden, 128) + _round_up(NC, 128)) * 4
    vmem_limit = min(2 * x_tile_bytes + 4 * weight_bytes
                     + TB * _round_up(NC, 128) * 4 + (8 << 20), 100 << 20)

    cost = pl.CostEstimate(
        flops=B_pad * C * HW + 2 * B_pad * (C * hidden + hidden * NC),
        transcendentals=0,
        bytes_accessed=(B_pad * C * HW * 4 + weight_bytes + B_pad * NC * 4),
    )

    out = pl.pallas_call(
        functools.partial(_head_kernel, inv_hw=1.0 / float(HW)),
        out_shape=jax.ShapeDtypeStruct((B_pad, NC), jnp.float32),
        grid=(n_b,),
        in_specs=[
            pl.BlockSpec((TB, HW, C), lambda i: (i, 0, 0)),
            pl.BlockSpec((hidden, C), lambda i: (0, 0)),
            pl.BlockSpec((1, hidden), lambda i: (0, 0)),
            pl.BlockSpec((NC, hidden), lambda i: (0, 0)),
            pl.BlockSpec((1, NC), lambda i: (0, 0)),
        ],
        out_specs=pl.BlockSpec((TB, NC), lambda i: (i, 0)),
        compiler_params=pltpu.CompilerParams(
            dimension_semantics=("parallel",),
            vmem_limit_bytes=vmem_limit,
        ),
        cost_estimate=cost,
    )(xr, w1, b1_row, w2, b2_row)

    return out[:B]
